# Initial kernel scaffold; baseline (speedup 1.0000x reference)
import jax
import jax.numpy as jnp
from jax.experimental import pallas as pl


def kernel(x, table):
    raise NotImplementedError("write your pallas kernel here")


# speedup vs baseline: 2.4138x; 2.4138x over previous
import functools

import jax
import jax.numpy as jnp
from jax import lax
from jax.experimental import pallas as pl
from jax.experimental.pallas import tpu as pltpu
from jax.experimental.pallas import tpu_sc as plsc

VOCAB_ = 100000
D = 128
B_FLAT = 4096 * 50
NW = 32
PER_W = B_FLAT // NW
CHUNK = 128
N_CHUNKS = PER_W // CHUNK
SCALE = float(D) ** 0.5


def _make_sc_kernel():
    mesh = plsc.VectorSubcoreMesh(core_axis_name="c", subcore_axis_name="s")

    @functools.partial(
        pl.kernel,
        mesh=mesh,
        out_type=jax.ShapeDtypeStruct((B_FLAT, D), jnp.float32),
        scratch_types=[
            pltpu.VMEM((N_CHUNKS, CHUNK), jnp.int32),
            pltpu.VMEM((2, CHUNK, D), jnp.float32),
            pltpu.SemaphoreType.DMA((2,)),
            pltpu.SemaphoreType.DMA((2,)),
        ],
    )
    def emb_kernel(x_hbm, table_hbm, out_hbm, idx_v, buf, gsem, ssem):
        wid = lax.axis_index("s") * 2 + lax.axis_index("c")
        base = wid * PER_W

        pltpu.sync_copy(x_hbm.at[wid], idx_v)

        def scale_buf(b):
            def row(i, _):
                for j in range(D // 16):
                    sl = pl.ds(j * 16, 16)
                    buf[b, i, sl] = buf[b, i, sl] * SCALE
                return 0
            lax.fori_loop(0, CHUNK, row, 0, unroll=2)

        def body(c, _):
            pltpu.async_copy(
                table_hbm.at[idx_v.at[c]], buf.at[0], gsem.at[0]
            ).wait()
            scale_buf(0)
            pltpu.sync_copy(
                buf.at[0], out_hbm.at[pl.ds(base + c * CHUNK, CHUNK)]
            )
            return 0

        lax.fori_loop(0, N_CHUNKS, body, 0)

    return emb_kernel


_EMB_KERNEL = _make_sc_kernel()


def kernel(x, table):
    x_rows = x.reshape(NW, N_CHUNKS, CHUNK).astype(jnp.int32)
    out = _EMB_KERNEL(x_rows, table)
    return out.reshape(x.shape[0], x.shape[1], D)


# speedup vs baseline: 2.8918x; 1.1980x over previous
import functools

import jax
import jax.numpy as jnp
from jax import lax
from jax.experimental import pallas as pl
from jax.experimental.pallas import tpu as pltpu
from jax.experimental.pallas import tpu_sc as plsc

VOCAB_ = 100000
D = 128
B_FLAT = 4096 * 50
NW = 32
PER_W = B_FLAT // NW
CHUNK = 128
N_CHUNKS = PER_W // CHUNK
NBUF = 5
SCALE = float(D) ** 0.5


def _make_sc_kernel():
    mesh = plsc.VectorSubcoreMesh(core_axis_name="c", subcore_axis_name="s")

    @functools.partial(
        pl.kernel,
        mesh=mesh,
        out_type=jax.ShapeDtypeStruct((B_FLAT, D), jnp.float32),
        scratch_types=[
            pltpu.VMEM((N_CHUNKS, CHUNK), jnp.int32),
            pltpu.VMEM((NBUF, CHUNK, D), jnp.float32),
            pltpu.SemaphoreType.DMA((NBUF,)),
            pltpu.SemaphoreType.DMA((NBUF,)),
        ],
    )
    def emb_kernel(x_hbm, table_hbm, out_hbm, idx_v, buf, gsem, ssem):
        wid = lax.axis_index("s") * 2 + lax.axis_index("c")
        base = wid * PER_W

        pltpu.sync_copy(x_hbm.at[wid], idx_v)

        def scale_buf(b):
            def row(i, _):
                for j in range(D // 16):
                    sl = pl.ds(j * 16, 16)
                    buf[b, i, sl] = buf[b, i, sl] * SCALE
                return 0
            lax.fori_loop(0, CHUNK, row, 0, unroll=2)

        def body(g, _):
            c0 = g * NBUF
            gathers = [
                pltpu.async_copy(
                    table_hbm.at[idx_v.at[c0 + j]], buf.at[j], gsem.at[j]
                )
                for j in range(NBUF)
            ]
            stores = []
            for j in range(NBUF):
                gathers[j].wait()
                scale_buf(j)
                stores.append(
                    pltpu.async_copy(
                        buf.at[j],
                        out_hbm.at[pl.ds(base + (c0 + j) * CHUNK, CHUNK)],
                        ssem.at[j],
                    )
                )
            for st in stores:
                st.wait()
            return 0

        lax.fori_loop(0, N_CHUNKS // NBUF, body, 0)

    return emb_kernel


_EMB_KERNEL = _make_sc_kernel()


def kernel(x, table):
    x_rows = x.reshape(NW, N_CHUNKS, CHUNK).astype(jnp.int32)
    out = _EMB_KERNEL(x_rows, table)
    return out.reshape(x.shape[0], x.shape[1], D)


# speedup vs baseline: 4.6589x; 1.6110x over previous
import functools

import jax
import jax.numpy as jnp
from jax import lax
from jax.experimental import pallas as pl
from jax.experimental.pallas import tpu as pltpu
from jax.experimental.pallas import tpu_sc as plsc

D = 128
BATCH = 4096
SEQ = 50
NW = 32
PER_W = BATCH // NW
NBUF = 4
SCALE = float(D) ** 0.5


def _make_sc_kernel():
    mesh = plsc.VectorSubcoreMesh(core_axis_name="c", subcore_axis_name="s")

    @functools.partial(
        pl.kernel,
        mesh=mesh,
        out_type=jax.ShapeDtypeStruct((BATCH, SEQ, D), jnp.float32),
        scratch_types=[
            pltpu.VMEM((PER_W, SEQ), jnp.int32),
            pltpu.VMEM((NBUF, SEQ, D), jnp.float32),
            pltpu.SemaphoreType.DMA((NBUF,)),
            pltpu.SemaphoreType.DMA((NBUF,)),
        ],
    )
    def emb_kernel(x_hbm, table_hbm, out_hbm, idx_v, buf, gsem, ssem):
        wid = lax.axis_index("s") * 2 + lax.axis_index("c")
        base = wid * PER_W

        pltpu.sync_copy(x_hbm.at[wid], idx_v)

        def scale_buf(b):
            def row(i, _):
                for j in range(D // 16):
                    sl = pl.ds(j * 16, 16)
                    buf[b, i, sl] = buf[b, i, sl] * SCALE
                return 0
            lax.fori_loop(0, SEQ, row, 0, unroll=2)

        def body(g, _):
            c0 = g * NBUF
            gathers = [
                pltpu.async_copy(
                    table_hbm.at[idx_v.at[c0 + j]], buf.at[j], gsem.at[j]
                )
                for j in range(NBUF)
            ]
            stores = []
            for j in range(NBUF):
                gathers[j].wait()
                scale_buf(j)
                stores.append(
                    pltpu.async_copy(
                        buf.at[j], out_hbm.at[base + c0 + j], ssem.at[j]
                    )
                )
            for st in stores:
                st.wait()
            return 0

        lax.fori_loop(0, PER_W // NBUF, body, 0)

    return emb_kernel


_EMB_KERNEL = _make_sc_kernel()


def kernel(x, table):
    x_rows = x.reshape(NW, PER_W, SEQ).astype(jnp.int32)
    return _EMB_KERNEL(x_rows, table)


# speedup vs baseline: 8.6403x; 1.8546x over previous
import functools

import jax
import jax.numpy as jnp
from jax import lax
from jax.experimental import pallas as pl
from jax.experimental.pallas import tpu as pltpu
from jax.experimental.pallas import tpu_sc as plsc

D = 128
BATCH = 4096
SEQ = 50
B_FLAT = BATCH * SEQ
NW = 32
PER_W = B_FLAT // NW
CHUNK = 128
N_CHUNKS = PER_W // CHUNK
NBUF = 5
SCALE = float(D) ** 0.5


def _make_sc_kernel():
    mesh = plsc.VectorSubcoreMesh(core_axis_name="c", subcore_axis_name="s")

    @functools.partial(
        pl.kernel,
        mesh=mesh,
        out_type=jax.ShapeDtypeStruct((B_FLAT, D), jnp.float32),
        scratch_types=[
            pltpu.VMEM((N_CHUNKS, CHUNK), jnp.int32),
            pltpu.VMEM((NBUF, CHUNK, D), jnp.float32),
            pltpu.SemaphoreType.DMA((NBUF,)),
            pltpu.SemaphoreType.DMA((NBUF,)),
        ],
    )
    def emb_kernel(x_hbm, table_hbm, out_hbm, idx_v, buf, gsem, ssem):
        wid = lax.axis_index("s") * 2 + lax.axis_index("c")
        base = wid * PER_W

        pltpu.sync_copy(x_hbm.at[wid], idx_v)

        def scale_buf(b):
            def row(i, _):
                for j in range(D // 16):
                    sl = pl.ds(j * 16, 16)
                    buf[b, i, sl] = buf[b, i, sl] * SCALE
                return 0
            lax.fori_loop(0, CHUNK, row, 0, unroll=2)

        def body(g, _):
            c0 = g * NBUF
            gathers = [
                pltpu.async_copy(
                    table_hbm.at[idx_v.at[c0 + j]], buf.at[j], gsem.at[j]
                )
                for j in range(NBUF)
            ]
            stores = []
            for j in range(NBUF):
                gathers[j].wait()
                scale_buf(j)
                stores.append(
                    pltpu.async_copy(
                        buf.at[j],
                        out_hbm.at[pl.ds(base + (c0 + j) * CHUNK, CHUNK)],
                        ssem.at[j],
                    )
                )
            for st in stores:
                st.wait()
            return 0

        lax.fori_loop(0, N_CHUNKS // NBUF, body, 0)

    return emb_kernel


_EMB_KERNEL = _make_sc_kernel()


def kernel(x, table):
    x_rows = x.T.reshape(NW, N_CHUNKS, CHUNK).astype(jnp.int32)
    out = _EMB_KERNEL(x_rows, table)
    return out.reshape(SEQ, BATCH, D).transpose(1, 0, 2)


# speedup vs baseline: 8.9712x; 1.0383x over previous
import functools

import jax
import jax.numpy as jnp
from jax import lax
from jax.experimental import pallas as pl
from jax.experimental.pallas import tpu as pltpu
from jax.experimental.pallas import tpu_sc as plsc

D = 128
BATCH = 4096
SEQ = 50
B_FLAT = BATCH * SEQ
NW = 32
PER_W = B_FLAT // NW
CHUNK = 128
N_CHUNKS = PER_W // CHUNK
NBUF = 5
SCALE = float(D) ** 0.5


def _make_sc_kernel():
    mesh = plsc.VectorSubcoreMesh(core_axis_name="c", subcore_axis_name="s")

    @functools.partial(
        pl.kernel,
        mesh=mesh,
        out_type=jax.ShapeDtypeStruct((B_FLAT, D), jnp.float32),
        scratch_types=[
            pltpu.VMEM((N_CHUNKS, CHUNK), jnp.int32),
            pltpu.VMEM((NBUF, CHUNK, D), jnp.float32),
            pltpu.SemaphoreType.DMA((NBUF,)),
            pltpu.SemaphoreType.DMA((NBUF,)),
        ],
    )
    def emb_kernel(x_hbm, table_hbm, out_hbm, idx_v, buf, gsem, ssem):
        wid = lax.axis_index("s") * 2 + lax.axis_index("c")
        base = wid * PER_W

        pltpu.sync_copy(x_hbm.at[wid], idx_v)

        def scale_buf(b):
            def row(i, _):
                for j in range(D // 16):
                    sl = pl.ds(j * 16, 16)
                    buf[b, i, sl] = buf[b, i, sl] * SCALE
                return 0
            lax.fori_loop(0, CHUNK, row, 0, unroll=4)

        def gather_start(c, b):
            pltpu.async_copy(table_hbm.at[idx_v.at[c]], buf.at[b], gsem.at[b])

        def gather_wait(c, b):
            pltpu.make_async_copy(
                table_hbm.at[idx_v.at[c]], buf.at[b], gsem.at[b]
            ).wait()

        def store_start(c, b):
            return pltpu.async_copy(
                buf.at[b], out_hbm.at[pl.ds(base + c * CHUNK, CHUNK)],
                ssem.at[b],
            )

        NG = N_CHUNKS // NBUF
        for j in range(NBUF):
            gather_start(j, j)

        def body(g, _):
            c0 = g * NBUF
            stores = []
            for j in range(NBUF):
                gather_wait(c0 + j, j)
                scale_buf(j)
                stores.append(store_start(c0 + j, j))
                if j >= 1:
                    stores[j - 1].wait()
                    gather_start(c0 + NBUF + j - 1, j - 1)
            stores[NBUF - 1].wait()
            gather_start(c0 + 2 * NBUF - 1, NBUF - 1)
            return 0

        lax.fori_loop(0, NG - 1, body, 0)

        c0 = (NG - 1) * NBUF
        last_stores = []
        for j in range(NBUF):
            gather_wait(c0 + j, j)
            scale_buf(j)
            last_stores.append(store_start(c0 + j, j))
        for st in last_stores:
            st.wait()

    return emb_kernel


_EMB_KERNEL = _make_sc_kernel()


def kernel(x, table):
    x_rows = x.T.reshape(NW, N_CHUNKS, CHUNK).astype(jnp.int32)
    out = _EMB_KERNEL(x_rows, table)
    return out.reshape(SEQ, BATCH, D).transpose(1, 0, 2)


# speedup vs baseline: 8.9726x; 1.0002x over previous
import functools

import jax
import jax.numpy as jnp
from jax import lax
from jax.experimental import pallas as pl
from jax.experimental.pallas import tpu as pltpu
from jax.experimental.pallas import tpu_sc as plsc

D = 128
BATCH = 4096
SEQ = 50
B_FLAT = BATCH * SEQ
NW = 32
PER_W = B_FLAT // NW
CHUNK = 128
N_CHUNKS = PER_W // CHUNK
NBUF = 5
SCALE = float(D) ** 0.5


def _make_sc_kernel():
    mesh = plsc.VectorSubcoreMesh(core_axis_name="c", subcore_axis_name="s")

    @functools.partial(
        pl.kernel,
        mesh=mesh,
        out_type=jax.ShapeDtypeStruct((B_FLAT, D), jnp.float32),
        scratch_types=[
            pltpu.VMEM((N_CHUNKS, CHUNK), jnp.int32),
            pltpu.VMEM((NBUF, CHUNK, D), jnp.float32),
            pltpu.SemaphoreType.DMA((NBUF,)),
            pltpu.SemaphoreType.DMA((NBUF,)),
            pltpu.SemaphoreType.DMA,
        ],
    )
    def emb_kernel(x_hbm, table_hbm, out_hbm, idx_v, buf, gsem, ssem, isem):
        wid = lax.axis_index("s") * 2 + lax.axis_index("c")
        base = wid * PER_W

        pltpu.sync_copy(x_hbm.at[wid, pl.ds(0, 8)], idx_v.at[pl.ds(0, 8)])
        idx_tail = pltpu.async_copy(
            x_hbm.at[wid, pl.ds(8, N_CHUNKS - 8)],
            idx_v.at[pl.ds(8, N_CHUNKS - 8)],
            isem,
        )

        def scale_buf(b):
            def row(i, _):
                for j in range(D // 16):
                    sl = pl.ds(j * 16, 16)
                    buf[b, i, sl] = buf[b, i, sl] * SCALE
                return 0
            lax.fori_loop(0, CHUNK, row, 0, unroll=4)

        def gather_start(c, b):
            pltpu.async_copy(table_hbm.at[idx_v.at[c]], buf.at[b], gsem.at[b])

        def gather_wait(c, b):
            pltpu.make_async_copy(
                table_hbm.at[idx_v.at[c]], buf.at[b], gsem.at[b]
            ).wait()

        def store_start(c, b):
            return pltpu.async_copy(
                buf.at[b], out_hbm.at[pl.ds(base + c * CHUNK, CHUNK)],
                ssem.at[b],
            )

        NG = N_CHUNKS // NBUF
        for j in range(NBUF):
            gather_start(j, j)
        idx_tail.wait()

        def body(g, _):
            c0 = g * NBUF
            stores = []
            for j in range(NBUF):
                gather_wait(c0 + j, j)
                scale_buf(j)
                stores.append(store_start(c0 + j, j))
                if j >= 1:
                    stores[j - 1].wait()
                    gather_start(c0 + NBUF + j - 1, j - 1)
            stores[NBUF - 1].wait()
            gather_start(c0 + 2 * NBUF - 1, NBUF - 1)
            return 0

        lax.fori_loop(0, NG - 1, body, 0)

        c0 = (NG - 1) * NBUF
        last_stores = []
        for j in range(NBUF):
            gather_wait(c0 + j, j)
            scale_buf(j)
            last_stores.append(store_start(c0 + j, j))
        for st in last_stores:
            st.wait()

    return emb_kernel


_EMB_KERNEL = _make_sc_kernel()


def kernel(x, table):
    x_rows = x.T.reshape(NW, N_CHUNKS, CHUNK).astype(jnp.int32)
    out = _EMB_KERNEL(x_rows, table)
    return out.reshape(SEQ, BATCH, D).transpose(1, 0, 2)
